# Initial kernel scaffold; baseline (speedup 1.0000x reference)
import jax
import jax.numpy as jnp
from jax.experimental import pallas as pl


def kernel(nodes, to_neighs, u2e_weight):
    raise NotImplementedError("write your pallas kernel here")


# speedup vs baseline: 1.6686x; 1.6686x over previous
import functools

import jax
import jax.numpy as jnp
import numpy as np
from jax import lax
from jax.experimental import pallas as pl
from jax.experimental.pallas import tpu as pltpu
from jax.experimental.pallas import tpu_sc as plsc

NC = 2
NS = 16
NW = NC * NS
LANES = 16
ROWS_PER_STREAM = 128
NBUF = 4


def _build_sc_call(n_pad, deg, emb, table_rows, npw):
    nodes_per_stream = ROWS_PER_STREAM // deg
    nchunk = (npw * deg) // ROWS_PER_STREAM
    ngroup = nchunk // NBUF
    nvec = emb // LANES
    inv_deg = np.float32(1.0 / deg)

    mesh = plsc.VectorSubcoreMesh(
        core_axis_name="c", subcore_axis_name="s",
        num_cores=NC, num_subcores=NS)

    @functools.partial(
        pl.kernel,
        out_type=jax.ShapeDtypeStruct((n_pad, emb), jnp.float32),
        mesh=mesh,
        scratch_types=(
            [pltpu.VMEM((nchunk, ROWS_PER_STREAM), jnp.int32),
             pltpu.VMEM((npw, emb), jnp.float32)]
            + [pltpu.VMEM((ROWS_PER_STREAM, emb), jnp.float32)] * NBUF
            + [pltpu.SemaphoreType.DMA] * NBUF
        ),
    )
    def sc_call(idx_hbm, table_hbm, out_hbm, idx_v, out_v, *rest):
        bufs, sems = rest[:NBUF], rest[NBUF:]
        i32 = np.int32
        w = lax.axis_index("s") * i32(NC) + lax.axis_index("c")

        pltpu.sync_copy(idx_hbm.at[w], idx_v)

        def gather_start(j, b):
            pltpu.async_copy(table_hbm.at[idx_v.at[j]], bufs[b], sems[b])

        def gather_wait(j, b):
            pltpu.make_async_copy(
                table_hbm.at[idx_v.at[j]], bufs[b], sems[b]).wait()

        for b in range(NBUF):
            gather_start(jnp.int32(b), b)

        def group_body(g, carry):
            for b in range(NBUF):
                j = g * i32(NBUF) + i32(b)
                gather_wait(j, b)
                buf = bufs[b]

                def node_body(n, c, buf=buf, j=j):
                    r = j * i32(nodes_per_stream) + n
                    base = n * i32(deg)
                    for v in range(nvec):
                        sl = pl.ds(v * LANES, LANES)
                        a = buf[base, sl]
                        for d in range(1, deg):
                            a = a + buf[base + i32(d), sl]
                        out_v[r, sl] = a * inv_deg
                    return c

                lax.fori_loop(i32(0), i32(nodes_per_stream), node_body, 0)

                jn = j + i32(NBUF)

                @pl.when(jn < i32(nchunk))
                def _(jn=jn, b=b):
                    gather_start(jn, b)
            return carry

        lax.fori_loop(i32(0), i32(ngroup), group_body, 0)

        pltpu.sync_copy(out_v, out_hbm.at[pl.ds(w * i32(npw), npw)])

    return sc_call


def kernel(nodes, to_neighs, u2e_weight):
    del nodes
    n, deg = to_neighs.shape
    table_rows, emb = u2e_weight.shape

    nodes_per_stream = ROWS_PER_STREAM // deg
    quantum = nodes_per_stream * NBUF
    npw = ((n + NW - 1) // NW + quantum - 1) // quantum * quantum
    n_pad = npw * NW

    with jax.enable_x64(False):
        idx = to_neighs.astype(jnp.int32).reshape(-1)
        idx = jnp.pad(idx, (0, n_pad * deg - n * deg))
        idx3 = idx.reshape(NW, (npw * deg) // ROWS_PER_STREAM,
                           ROWS_PER_STREAM)

        table = u2e_weight.astype(jnp.float32)
        sc_call = _build_sc_call(n_pad, deg, emb, table_rows, npw)
        out = sc_call(idx3, table)
        return out[:n]
